# baseline (speedup 1.0000x reference)
import functools

import jax
import jax.numpy as jnp
from jax import lax
from jax.experimental import pallas as pl
from jax.experimental.pallas import tpu as pltpu
from jax.experimental.pallas import tpu_sc as plsc

NC = 2
NS = 16
L = 16
NW = NC * NS
R = 16
NSLOT = 3


def _sc_body(T, D, TPW, input_hbm, mask_hbm, source_hbm, out_hbm,
             mask_p, mask_v, idx_m, idx_u,
             idx_g0, idx_g1, idx_g2, idx_s0, idx_s1, idx_s2, buf,
             sem_g0, sem_g1, sem_g2, sem_s0, sem_s1, sem_s2):
    idx_gs = (idx_g0, idx_g1, idx_g2)
    idx_ss = (idx_s0, idx_s1, idx_s2)
    sem_gs = (sem_g0, sem_g1, sem_g2)
    sem_ss = (sem_s0, sem_s1, sem_s2)

    wid = lax.axis_index("s") * NC + lax.axis_index("c")
    base = wid * TPW
    lane = lax.broadcasted_iota(jnp.int32, (L,), 0)

    pltpu.sync_copy(mask_hbm, mask_p)

    for g in range(TPW // (4 * L)):
        w32 = mask_p[pl.ds(base // 4 + g * L, L)]
        for j in range(4):
            plsc.store_scatter(mask_v, [g * 4 * L + 4 * lane + j],
                               (w32 >> (8 * j)) & 0xFF)

    def psum():
        def psum_step(k, accs):
            return tuple(a + mask_p[pl.ds((k * 8 + u) * L, L)]
                         for u, a in enumerate(accs))
        accs = lax.fori_loop(0, base // (8 * 4 * L), psum_step,
                             tuple(jnp.zeros((L,), jnp.int32)
                                   for _ in range(8)))
        s = sum(accs)
        s = (s & 0xFF) + ((s >> 8) & 0xFF) + ((s >> 16) & 0xFF) + \
            ((s >> 24) & 0xFF)
        return jnp.sum(s)

    def build_step(i, carry):
        cnt_m, cnt_u, last_m, last_u = carry
        toks = base + i * L + lane
        m = mask_v[pl.ds(i * L, L)]
        mb = m > 0
        incl = plsc.cumsum(m)
        n_m = jnp.sum(m)
        pos_m = cnt_m + incl - 1
        plsc.store_scatter(idx_m, [pos_m], toks, mask=mb)
        incl_u = lane + 1 - incl
        pos_u = cnt_u + incl_u - 1
        plsc.store_scatter(idx_u, [pos_u], toks, mask=jnp.logical_not(mb))
        last_m = jnp.maximum(last_m, jnp.max(jnp.where(mb, toks, -1)))
        last_u = jnp.maximum(last_u, jnp.max(jnp.where(mb, -1, toks)))
        return cnt_m + n_m, cnt_u + (L - n_m), last_m, last_u

    km, ku, last_m, last_u = lax.fori_loop(
        0, TPW // L, build_step,
        (jnp.int32(0), jnp.int32(0), jnp.int32(-1), jnp.int32(-1)))

    @pl.when((km & (L - 1)) != 0)
    def _():
        p = (km & ~(L - 1)) + lane
        plsc.store_scatter(idx_m, [p], jnp.full((L,), last_m, jnp.int32),
                           mask=p >= km)

    @pl.when((ku & (L - 1)) != 0)
    def _():
        p = (ku & ~(L - 1)) + lane
        plsc.store_scatter(idx_u, [p], jnp.full((L,), last_u, jnp.int32),
                           mask=p >= ku)

    def run_path(src_hbm, nch, make_gidx, make_sidx, after_prime=None):
        def start_gather(c, s):
            idx_gs[s][...] = make_gidx(c)
            pltpu.make_async_copy(src_hbm.at[idx_gs[s]],
                                  buf.at[pl.ds(s * R, R)], sem_gs[s]).start()

        def wait_gather(s):
            pltpu.make_async_copy(src_hbm.at[idx_gs[s]],
                                  buf.at[pl.ds(s * R, R)], sem_gs[s]).wait()

        def start_scatter(c, s):
            idx_ss[s][...] = make_sidx(c)
            pltpu.make_async_copy(buf.at[pl.ds(s * R, R)],
                                  out_hbm.at[idx_ss[s]], sem_ss[s]).start()

        def wait_scatter(s):
            pltpu.make_async_copy(buf.at[pl.ds(s * R, R)],
                                  out_hbm.at[idx_ss[s]], sem_ss[s]).wait()

        @pl.when(nch > 0)
        def _():
            start_gather(0, 0)

        @pl.when(nch > 1)
        def _():
            start_gather(1, 1)

        res = after_prime() if after_prime is not None else None

        def triple(c3, _):
            for b in range(NSLOT):
                c = c3 * NSLOT + b
                s2 = (b + 2) % NSLOT

                @pl.when(c < nch)
                def _():
                    wait_gather(b)
                    start_scatter(c, b)

                @pl.when((c >= 1) & (c <= nch - 1))
                def _():
                    wait_scatter(s2)

                @pl.when(c + 2 < nch)
                def _():
                    start_gather(c + 2, s2)
            return 0

        lax.fori_loop(0, (nch + NSLOT - 1) // NSLOT, triple, 0)

        for s in range(NSLOT):
            @pl.when((nch > 0) & ((nch - 1) % NSLOT == s))
            def _():
                wait_scatter(s)

        return res

    c_start = run_path(
        input_hbm, (ku + R - 1) // R,
        lambda c: idx_u[pl.ds(c * R, R)],
        lambda c: idx_u[pl.ds(c * R, R)],
        after_prime=psum)

    run_path(
        source_hbm, (km + R - 1) // R,
        lambda c: jnp.minimum(c_start + c * R + lane, c_start + km - 1),
        lambda c: idx_m[pl.ds(c * R, R)])


@functools.partial(jax.jit, static_argnames=("T", "D"))
def _masked_scatter_rows(input_2d, mask32, source_2d, *, T, D):
    TPW = T // NW
    mesh = plsc.VectorSubcoreMesh(core_axis_name="c", subcore_axis_name="s",
                                  num_cores=NC, num_subcores=NS)
    body = functools.partial(_sc_body, T, D, TPW)
    return pl.kernel(
        body,
        out_type=jax.ShapeDtypeStruct((T, D), jnp.float32),
        mesh=mesh,
        compiler_params=pltpu.CompilerParams(needs_layout_passes=False),
        scratch_types=[
            pltpu.VMEM((T // 4,), jnp.int32),
            pltpu.VMEM((TPW,), jnp.int32),
            pltpu.VMEM((TPW,), jnp.int32),
            pltpu.VMEM((TPW,), jnp.int32),
            pltpu.VMEM((R,), jnp.int32),
            pltpu.VMEM((R,), jnp.int32),
            pltpu.VMEM((R,), jnp.int32),
            pltpu.VMEM((R,), jnp.int32),
            pltpu.VMEM((R,), jnp.int32),
            pltpu.VMEM((R,), jnp.int32),
            pltpu.VMEM((NSLOT * R, D), jnp.float32),
            pltpu.SemaphoreType.DMA,
            pltpu.SemaphoreType.DMA,
            pltpu.SemaphoreType.DMA,
            pltpu.SemaphoreType.DMA,
            pltpu.SemaphoreType.DMA,
            pltpu.SemaphoreType.DMA,
        ],
    )(input_2d, mask32, source_2d)


def kernel(inputs_embeds, mask_1d, source):
    B, S, D = inputs_embeds.shape
    T = B * S
    assert T % (NW * 8 * L) == 0
    input_2d = inputs_embeds.reshape(T, D)
    source_2d = source.reshape(T, D)
    mask_p = mask_1d.reshape(T).view(jnp.uint8).view(jnp.int32)
    out = _masked_scatter_rows(input_2d, mask_p, source_2d, T=T, D=D)
    return out.reshape(B, S, D)


# speedup vs baseline: 1.0109x; 1.0109x over previous
import functools

import jax
import jax.numpy as jnp
from jax import lax
from jax.experimental import pallas as pl
from jax.experimental.pallas import tpu as pltpu
from jax.experimental.pallas import tpu_sc as plsc

NC = 2
NS = 16
L = 16
NW = NC * NS
R = 16
NSLOT = 3


def _sc_body(T, D, TPW, input_hbm, mask_hbm, source_hbm, out_hbm,
             mask_p, mask_v, idx_m, idx_u,
             idx_g0, idx_g1, idx_g2, idx_s0, idx_s1, idx_s2, buf,
             sem_g0, sem_g1, sem_g2, sem_s0, sem_s1, sem_s2):
    idx_gs = (idx_g0, idx_g1, idx_g2)
    idx_ss = (idx_s0, idx_s1, idx_s2)
    sem_gs = (sem_g0, sem_g1, sem_g2)
    sem_ss = (sem_s0, sem_s1, sem_s2)

    wid = lax.axis_index("s") * NC + lax.axis_index("c")
    base = wid * TPW
    lane = lax.broadcasted_iota(jnp.int32, (L,), 0)

    pltpu.sync_copy(mask_hbm, mask_p)

    for g in range(TPW // (4 * L)):
        w32 = mask_p[pl.ds(base // 4 + g * L, L)]
        for j in range(4):
            plsc.store_scatter(mask_v, [g * 4 * L + 4 * lane + j],
                               (w32 >> (8 * j)) & 0xFF)

    def psum():
        def psum_step(k, accs):
            return tuple(a + mask_p[pl.ds((k * 8 + u) * L, L)]
                         for u, a in enumerate(accs))
        accs = lax.fori_loop(0, base // (8 * 4 * L), psum_step,
                             tuple(jnp.zeros((L,), jnp.int32)
                                   for _ in range(8)))
        s = sum(accs)
        s = (s & 0xFF) + ((s >> 8) & 0xFF) + ((s >> 16) & 0xFF) + \
            ((s >> 24) & 0xFF)
        return jnp.sum(s)

    def build_step(i, carry):
        cnt_m, cnt_u, last_m, last_u = carry
        toks = base + i * L + lane
        m = mask_v[pl.ds(i * L, L)]
        mb = m > 0
        incl = plsc.cumsum(m)
        n_m = jnp.sum(m)
        pos_m = cnt_m + incl - 1
        plsc.store_scatter(idx_m, [pos_m], toks, mask=mb)
        incl_u = lane + 1 - incl
        pos_u = cnt_u + incl_u - 1
        plsc.store_scatter(idx_u, [pos_u], toks, mask=jnp.logical_not(mb))
        last_m = jnp.maximum(last_m, jnp.max(jnp.where(mb, toks, -1)))
        last_u = jnp.maximum(last_u, jnp.max(jnp.where(mb, -1, toks)))
        return cnt_m + n_m, cnt_u + (L - n_m), last_m, last_u

    km, ku, last_m, last_u = lax.fori_loop(
        0, TPW // L, build_step,
        (jnp.int32(0), jnp.int32(0), jnp.int32(-1), jnp.int32(-1)))

    @pl.when((km & (L - 1)) != 0)
    def _():
        p = (km & ~(L - 1)) + lane
        plsc.store_scatter(idx_m, [p], jnp.full((L,), last_m, jnp.int32),
                           mask=p >= km)

    @pl.when((ku & (L - 1)) != 0)
    def _():
        p = (ku & ~(L - 1)) + lane
        plsc.store_scatter(idx_u, [p], jnp.full((L,), last_u, jnp.int32),
                           mask=p >= ku)

    c_start = psum()

    nch_u = (ku + R - 1) // R
    nch_m = (km + R - 1) // R
    nch = nch_u + nch_m

    def bslot(s):
        return buf.at[pl.ds(s * R, R)]

    def start_gather(c, s):
        @pl.when(c < nch_u)
        def _():
            idx_gs[s][...] = idx_u[pl.ds(c * R, R)]
            pltpu.make_async_copy(input_hbm.at[idx_gs[s]], bslot(s),
                                  sem_gs[s]).start()

        @pl.when(c >= nch_u)
        def _():
            cm = c - nch_u
            idx_gs[s][...] = jnp.minimum(c_start + cm * R + lane,
                                         c_start + km - 1)
            pltpu.make_async_copy(source_hbm.at[idx_gs[s]], bslot(s),
                                  sem_gs[s]).start()

    def wait_gather(c, s):
        @pl.when(c < nch_u)
        def _():
            pltpu.make_async_copy(input_hbm.at[idx_gs[s]], bslot(s),
                                  sem_gs[s]).wait()

        @pl.when(c >= nch_u)
        def _():
            pltpu.make_async_copy(source_hbm.at[idx_gs[s]], bslot(s),
                                  sem_gs[s]).wait()

    def start_scatter(c, s):
        @pl.when(c < nch_u)
        def _():
            idx_ss[s][...] = idx_u[pl.ds(c * R, R)]

        @pl.when(c >= nch_u)
        def _():
            idx_ss[s][...] = idx_m[pl.ds((c - nch_u) * R, R)]

        pltpu.make_async_copy(bslot(s), out_hbm.at[idx_ss[s]],
                              sem_ss[s]).start()

    def wait_scatter(s):
        pltpu.make_async_copy(bslot(s), out_hbm.at[idx_ss[s]],
                              sem_ss[s]).wait()

    start_gather(0, 0)

    @pl.when(nch > 1)
    def _():
        start_gather(1, 1)

    def triple(c3, _):
        for b in range(NSLOT):
            c = c3 * NSLOT + b
            s2 = (b + 2) % NSLOT

            @pl.when(c < nch)
            def _():
                wait_gather(c, b)
                start_scatter(c, b)

            @pl.when((c >= 1) & (c <= nch - 1))
            def _():
                wait_scatter(s2)

            @pl.when(c + 2 < nch)
            def _():
                start_gather(c + 2, s2)
        return 0

    lax.fori_loop(0, (nch + NSLOT - 1) // NSLOT, triple, 0)

    for s in range(NSLOT):
        @pl.when((nch - 1) % NSLOT == s)
        def _():
            wait_scatter(s)


@functools.partial(jax.jit, static_argnames=("T", "D"))
def _masked_scatter_rows(input_2d, mask32, source_2d, *, T, D):
    TPW = T // NW
    mesh = plsc.VectorSubcoreMesh(core_axis_name="c", subcore_axis_name="s",
                                  num_cores=NC, num_subcores=NS)
    body = functools.partial(_sc_body, T, D, TPW)
    return pl.kernel(
        body,
        out_type=jax.ShapeDtypeStruct((T, D), jnp.float32),
        mesh=mesh,
        compiler_params=pltpu.CompilerParams(needs_layout_passes=False),
        scratch_types=[
            pltpu.VMEM((T // 4,), jnp.int32),
            pltpu.VMEM((TPW,), jnp.int32),
            pltpu.VMEM((TPW,), jnp.int32),
            pltpu.VMEM((TPW,), jnp.int32),
            pltpu.VMEM((R,), jnp.int32),
            pltpu.VMEM((R,), jnp.int32),
            pltpu.VMEM((R,), jnp.int32),
            pltpu.VMEM((R,), jnp.int32),
            pltpu.VMEM((R,), jnp.int32),
            pltpu.VMEM((R,), jnp.int32),
            pltpu.VMEM((NSLOT * R, D), jnp.float32),
            pltpu.SemaphoreType.DMA,
            pltpu.SemaphoreType.DMA,
            pltpu.SemaphoreType.DMA,
            pltpu.SemaphoreType.DMA,
            pltpu.SemaphoreType.DMA,
            pltpu.SemaphoreType.DMA,
        ],
    )(input_2d, mask32, source_2d)


def kernel(inputs_embeds, mask_1d, source):
    B, S, D = inputs_embeds.shape
    T = B * S
    assert T % (NW * 8 * L) == 0
    input_2d = inputs_embeds.reshape(T, D)
    source_2d = source.reshape(T, D)
    mask_p = mask_1d.reshape(T).view(jnp.uint8).view(jnp.int32)
    out = _masked_scatter_rows(input_2d, mask_p, source_2d, T=T, D=D)
    return out.reshape(B, S, D)
